# baseline (speedup 1.0000x reference)
import functools

import jax
import jax.numpy as jnp
from jax import lax
from jax.experimental import pallas as pl
from jax.experimental.pallas import tpu as pltpu
from jax.experimental.pallas import tpu_sc as plsc


def _gather_kernel(n_total, dim, n_per_w, chunk, n_chunks, nc):
    mesh = plsc.VectorSubcoreMesh(core_axis_name="c", subcore_axis_name="s")

    @functools.partial(
        pl.kernel,
        mesh=mesh,
        out_type=jax.ShapeDtypeStruct((n_total, dim), jnp.float32),
        scratch_types=[
            pltpu.VMEM((n_per_w,), jnp.int32),
            pltpu.VMEM((chunk, dim), jnp.float32),
            pltpu.SemaphoreType.DMA,
        ],
        compiler_params=pltpu.CompilerParams(use_tc_tiling_on_sc=False),
    )
    def k(idx_hbm, table_hbm, out_hbm, idx_v, rows_v, gsem):
        wid = lax.axis_index("s") * nc + lax.axis_index("c")
        base = wid * n_per_w
        pltpu.sync_copy(idx_hbm.at[pl.ds(base, n_per_w)], idx_v)

        def body(g, _):
            off = g * chunk
            pltpu.async_copy(
                table_hbm.at[idx_v.at[pl.ds(off, chunk)]], rows_v, gsem
            ).wait()
            pltpu.sync_copy(rows_v, out_hbm.at[pl.ds(base + off, chunk)])
            return 0

        lax.fori_loop(0, n_chunks, body, 0)

    return k


def kernel(input_ids, table):
    b, s = input_ids.shape
    v, dim = table.shape
    n_total = b * s
    nw = 32
    nc = 2
    n_per_w = n_total // nw
    chunk = 512
    n_chunks = n_per_w // chunk
    assert n_per_w * nw == n_total and chunk * n_chunks == n_per_w

    idx = input_ids.reshape(n_total).astype(jnp.int32)
    out = _gather_kernel(n_total, dim, n_per_w, chunk, n_chunks, nc)(idx, table)
    return out.reshape(b, s, dim)


# speedup vs baseline: 1.0243x; 1.0243x over previous
import functools

import jax
import jax.numpy as jnp
from jax import lax
from jax.experimental import pallas as pl
from jax.experimental.pallas import tpu as pltpu
from jax.experimental.pallas import tpu_sc as plsc

_NBUF = 4


def _gather_kernel(n_total, dim, n_per_w, chunk, n_chunks, nc):
    mesh = plsc.VectorSubcoreMesh(core_axis_name="c", subcore_axis_name="s")
    n_iters = n_chunks // _NBUF

    @functools.partial(
        pl.kernel,
        mesh=mesh,
        out_type=jax.ShapeDtypeStruct((n_total, dim), jnp.float32),
        scratch_types=[
            pltpu.VMEM((n_per_w,), jnp.int32),
            pltpu.VMEM((_NBUF, chunk, dim), jnp.float32),
            pltpu.SemaphoreType.DMA,
            pltpu.SemaphoreType.DMA,
        ],
        compiler_params=pltpu.CompilerParams(use_tc_tiling_on_sc=False),
    )
    def k(idx_hbm, table_hbm, out_hbm, idx_v, rows_v, gsem, osem):
        wid = lax.axis_index("s") * nc + lax.axis_index("c")
        base = wid * n_per_w
        pltpu.sync_copy(idx_hbm.at[pl.ds(base, n_per_w)], idx_v)

        def fire_gather(g, b):
            pltpu.async_copy(
                table_hbm.at[idx_v.at[pl.ds(g * chunk, chunk)]],
                rows_v.at[b],
                gsem,
            )

        def fire_out(g, b):
            pltpu.async_copy(
                rows_v.at[b], out_hbm.at[pl.ds(base + g * chunk, chunk)], osem
            )

        def wait_gather(b):
            pltpu.make_async_copy(
                table_hbm.at[idx_v.at[pl.ds(0, chunk)]], rows_v.at[b], gsem
            ).wait()

        def wait_out(b):
            pltpu.make_async_copy(
                rows_v.at[b], out_hbm.at[pl.ds(base, chunk)], osem
            ).wait()

        for b in range(_NBUF - 1):
            fire_gather(b, b)

        def body(i, _):
            for b in range(_NBUF):
                g = i * _NBUF + b
                wait_gather(b)
                if b == 0:
                    @pl.when(i > 0)
                    def _():
                        wait_out(_NBUF - 1)
                    fire_gather(g + _NBUF - 1, _NBUF - 1)
                else:
                    wait_out(b - 1)

                    @pl.when(i < n_iters - 1)
                    def _():
                        fire_gather(g + _NBUF - 1, b - 1)
                fire_out(g, b)
            return 0

        lax.fori_loop(0, n_iters, body, 0)
        wait_out(_NBUF - 1)

    return k


def kernel(input_ids, table):
    b, s = input_ids.shape
    v, dim = table.shape
    n_total = b * s
    nw = 32
    nc = 2
    n_per_w = n_total // nw
    chunk = 400
    n_chunks = n_per_w // chunk
    assert n_per_w * nw == n_total
    assert chunk * n_chunks == n_per_w and n_chunks % _NBUF == 0

    idx = input_ids.reshape(n_total).astype(jnp.int32)
    out = _gather_kernel(n_total, dim, n_per_w, chunk, n_chunks, nc)(idx, table)
    return out.reshape(b, s, dim)
